# baseline (speedup 1.0000x reference)
import jax
import jax.numpy as jnp
from jax import lax
from jax.experimental import pallas as pl
from jax.experimental.pallas import tpu as pltpu

F32 = jnp.float32


def _enc1_body(x_ref, w_ref, b_ref, o_ref):
    acc = jnp.dot(x_ref[0], w_ref[:], preferred_element_type=F32)
    o_ref[0] = jnp.maximum(acc + b_ref[:], 0.0)


def _enc2_body(x_ref, w_ref, b_ref, o_ref):
    t = pl.program_id(1)
    acc = jnp.zeros((784, 32), F32)
    for kh in range(4):
        for kw in range(4):
            s = x_ref[0, kh % 2, kw % 2, pl.ds(14 * t + kh // 2, 14),
                      pl.ds(kw // 2, 56), :]
            acc = acc + jnp.dot(s.reshape(784, 64), w_ref[kh * 4 + kw],
                                preferred_element_type=F32)
    o_ref[0] = acc + b_ref[:]


def _vq_body(z_ref, c_ref, q_ref, i_ref, l_ref):
    z = z_ref[:]
    c = c_ref[:]
    zn = jnp.sum(z * z, axis=1, keepdims=True)
    cn = jnp.sum(c * c, axis=1)
    m = lax.dot_general(z, c, (((1,), (1,)), ((), ())),
                        preferred_element_type=F32)
    d = (zn + cn[None, :]) - 2.0 * m
    dmin = jnp.min(d, axis=1, keepdims=True)
    iota = lax.broadcasted_iota(jnp.int32, (256, 1024), 1)
    idx = jnp.min(jnp.where(d == dmin, iota, 1024), axis=1)
    onehot = (iota == idx[:, None]).astype(F32)
    q = jnp.dot(onehot, c, preferred_element_type=F32)
    q_ref[:] = q
    i_ref[0, 0, :] = idx
    diff = q - z
    ss = jnp.sum(diff * diff)
    @pl.when(pl.program_id(0) == 0)
    def _():
        l_ref[0, 0] = ss
    @pl.when(pl.program_id(0) != 0)
    def _():
        l_ref[0, 0] = l_ref[0, 0] + ss


def _dec1_body(q_ref, w1_ref, b1_ref, o_ref):
    a = pl.program_id(1)
    b = pl.program_id(2)
    t = pl.program_id(3)
    acc = jnp.zeros((784, 64), F32)
    for dh in range(2):
        for dw in range(2):
            u0 = (a + 2 * dh + 1) // 2
            v0 = (b + 2 * dw + 1) // 2
            s = q_ref[0, pl.ds(14 * t + u0, 14), pl.ds(v0, 56), :]
            w = w1_ref[a + 2 * dh, b + 2 * dw]
            acc = acc + jnp.dot(s.reshape(784, 32), w,
                                preferred_element_type=F32)
    o_ref[0, 0, 0] = jnp.maximum(acc + b1_ref[:], 0.0)


def _dec2_body(y_ref, w2_ref, b2_ref, o_ref):
    t = pl.program_id(1)
    p = jnp.zeros((3136, 4), F32)
    for u0 in range(3):
        for v0 in range(3):
            s = y_ref[0, pl.ds(28 * t + u0, 28), pl.ds(v0, 112), :]
            p = p + jnp.dot(s.reshape(3136, 64), w2_ref[u0, v0],
                            preferred_element_type=F32)
    o_ref[0] = jax.nn.sigmoid(p + b2_ref[:])


def kernel(x, enc1_w, enc1_b, enc2_w, enc2_b, codebook,
           dec1_w, dec1_b, dec2_w, dec2_b):
    B = 16
    xp = jnp.pad(x[:, 0], ((0, 0), (1, 1), (1, 1)))
    xpar = xp.reshape(B, 113, 2, 113, 2).transpose(0, 2, 4, 1, 3)
    taps = [xpar[:, kh % 2, kw % 2, kh // 2:kh // 2 + 112,
                 kw // 2:kw // 2 + 112]
            for kh in range(4) for kw in range(4)]
    xcol = jnp.stack(taps, axis=-1).reshape(B, 12544, 16)
    w1m = enc1_w[:, 0].reshape(64, 16).T
    z1 = pl.pallas_call(
        _enc1_body,
        grid=(B, 4),
        in_specs=[pl.BlockSpec((1, 3136, 16), lambda b, t: (b, t, 0)),
                  pl.BlockSpec((16, 64), lambda b, t: (0, 0)),
                  pl.BlockSpec((1, 64), lambda b, t: (0, 0))],
        out_specs=pl.BlockSpec((1, 3136, 64), lambda b, t: (b, t, 0)),
        out_shape=jax.ShapeDtypeStruct((B, 12544, 64), F32),
    )(xcol, w1m, enc1_b.reshape(1, 64))

    z1p = jnp.pad(z1.reshape(B, 112, 112, 64),
                  ((0, 0), (1, 1), (1, 1), (0, 0)))
    z1p = z1p.reshape(B, 57, 2, 57, 2, 64).transpose(0, 2, 4, 1, 3, 5)
    w2m = enc2_w.transpose(2, 3, 1, 0).reshape(16, 64, 32)
    zf = pl.pallas_call(
        _enc2_body,
        grid=(B, 4),
        in_specs=[pl.BlockSpec((1, 2, 2, 57, 57, 64),
                               lambda b, t: (b, 0, 0, 0, 0, 0)),
                  pl.BlockSpec((16, 64, 32), lambda b, t: (0, 0, 0)),
                  pl.BlockSpec((1, 32), lambda b, t: (0, 0))],
        out_specs=pl.BlockSpec((1, 784, 32), lambda b, t: (b, t, 0)),
        out_shape=jax.ShapeDtypeStruct((B, 3136, 32), F32),
    )(z1p, w2m, enc2_b.reshape(1, 32))
    zf = zf.reshape(50176, 32)

    nblk = 196
    q, idx, loss = pl.pallas_call(
        _vq_body,
        grid=(nblk,),
        in_specs=[pl.BlockSpec((256, 32), lambda i: (i, 0)),
                  pl.BlockSpec((1024, 32), lambda i: (0, 0))],
        out_specs=[pl.BlockSpec((256, 32), lambda i: (i, 0)),
                   pl.BlockSpec((1, 1, 256), lambda i: (i, 0, 0)),
                   pl.BlockSpec((1, 1), lambda i: (0, 0),
                                memory_space=pltpu.SMEM)],
        out_shape=[jax.ShapeDtypeStruct((50176, 32), F32),
                   jax.ShapeDtypeStruct((nblk, 1, 256), jnp.int32),
                   jax.ShapeDtypeStruct((1, 1), F32)],
    )(zf, codebook)
    indices = idx.reshape(B, 56, 56)
    vq_loss = (loss[0, 0] * (1.25 / (16 * 32 * 56 * 56))).reshape(())

    qp = jnp.pad(q.reshape(B, 56, 56, 32), ((0, 0), (1, 1), (1, 1), (0, 0)))
    wd1 = dec1_w.transpose(2, 3, 1, 0)
    yp = pl.pallas_call(
        _dec1_body,
        grid=(B, 2, 2, 4),
        in_specs=[pl.BlockSpec((1, 58, 58, 32),
                               lambda b, a, c, t: (b, 0, 0, 0)),
                  pl.BlockSpec((4, 4, 32, 64),
                               lambda b, a, c, t: (0, 0, 0, 0)),
                  pl.BlockSpec((1, 64), lambda b, a, c, t: (0, 0))],
        out_specs=pl.BlockSpec((1, 1, 1, 784, 64),
                               lambda b, a, c, t: (b, a, c, t, 0)),
        out_shape=jax.ShapeDtypeStruct((B, 2, 2, 3136, 64), F32),
    )(qp, wd1, dec1_b.reshape(1, 64))
    y = yp.reshape(B, 2, 2, 56, 56, 64).transpose(0, 3, 1, 4, 2, 5)
    y = y.reshape(B, 112, 112, 64)
    ypad = jnp.pad(y, ((0, 0), (1, 1), (1, 1), (0, 0)))

    wd2 = jnp.zeros((3, 3, 64, 4), F32)
    for kh in range(4):
        for kw in range(4):
            wd2 = wd2.at[(kh + 1) // 2, (kw + 1) // 2, :,
                         2 * (kh % 2) + (kw % 2)].set(dec2_w[0, :, kh, kw])
    recon_p = pl.pallas_call(
        _dec2_body,
        grid=(B, 4),
        in_specs=[pl.BlockSpec((1, 114, 114, 64),
                               lambda b, t: (b, 0, 0, 0)),
                  pl.BlockSpec((3, 3, 64, 4), lambda b, t: (0, 0, 0, 0)),
                  pl.BlockSpec((1, 1), lambda b, t: (0, 0))],
        out_specs=pl.BlockSpec((1, 3136, 4), lambda b, t: (b, t, 0)),
        out_shape=jax.ShapeDtypeStruct((B, 12544, 4), F32),
    )(ypad, wd2, dec2_b.reshape(1, 1))
    x_recon = recon_p.reshape(B, 112, 112, 2, 2).transpose(0, 1, 3, 2, 4)
    x_recon = x_recon.reshape(B, 1, 224, 224)
    return (x_recon, vq_loss, indices)


# speedup vs baseline: 1.2093x; 1.2093x over previous
import jax
import jax.numpy as jnp
from jax import lax
from jax.experimental import pallas as pl
from jax.experimental.pallas import tpu as pltpu

F32 = jnp.float32


def _zero_halo(ref):
    ref[0:1, :, :] = jnp.zeros((1, 114, 64), F32)
    ref[113:114, :, :] = jnp.zeros((1, 114, 64), F32)
    ref[:, 0:1, :] = jnp.zeros((114, 1, 64), F32)
    ref[:, 113:114, :] = jnp.zeros((114, 1, 64), F32)


def _enc_body(x_ref, w1_ref, b1_ref, w2_ref, b2_ref, o_ref, z1_ref):
    _zero_halo(z1_ref)
    for t in range(8):
        acc = jnp.dot(x_ref[0, pl.ds(1568 * t, 1568), :], w1_ref[:],
                      preferred_element_type=F32)
        acc = jnp.maximum(acc + b1_ref[:], 0.0)
        z1_ref[pl.ds(1 + 14 * t, 14), 1:113, :] = acc.reshape(14, 112, 64)
    for t in range(4):
        acc = jnp.zeros((784, 32), F32)
        for kh in range(4):
            for kw in range(4):
                s = z1_ref[pl.Slice(28 * t + kh, 14, 2),
                           pl.Slice(kw, 56, 2), :]
                acc = acc + jnp.dot(s.reshape(784, 64), w2_ref[kh * 4 + kw],
                                    preferred_element_type=F32)
        o_ref[0, pl.ds(784 * t, 784), :] = acc + b2_ref[:]


def _vq_body(z_ref, c_ref, q_ref, i_ref, l_ref):
    z = z_ref[:]
    c = c_ref[:]
    zn = jnp.sum(z * z, axis=1, keepdims=True)
    cn = jnp.sum(c * c, axis=1)
    m = lax.dot_general(z, c, (((1,), (1,)), ((), ())),
                        preferred_element_type=F32)
    d = (zn + cn[None, :]) - 2.0 * m
    dmin = jnp.min(d, axis=1, keepdims=True)
    iota = lax.broadcasted_iota(jnp.int32, (256, 1024), 1)
    idx = jnp.min(jnp.where(d == dmin, iota, 1024), axis=1)
    onehot = (iota == idx[:, None]).astype(F32)
    q = jnp.dot(onehot, c, preferred_element_type=F32)
    q_ref[:] = q
    i_ref[0, 0, :] = idx
    diff = q - z
    ss = jnp.sum(diff * diff)
    @pl.when(pl.program_id(0) == 0)
    def _():
        l_ref[0, 0] = ss
    @pl.when(pl.program_id(0) != 0)
    def _():
        l_ref[0, 0] = l_ref[0, 0] + ss


def _dec_body(q_ref, w1_ref, b1_ref, w2_ref, b2_ref, o_ref, y_ref):
    _zero_halo(y_ref)
    for a in range(2):
        for c in range(2):
            for t in range(4):
                acc = jnp.zeros((784, 64), F32)
                for dh in range(2):
                    for dw in range(2):
                        kh = a + 2 * dh
                        kw = c + 2 * dw
                        u0 = (kh + 1) // 2
                        v0 = (kw + 1) // 2
                        s = q_ref[0, pl.ds(14 * t + u0, 14),
                                  pl.ds(v0, 56), :]
                        acc = acc + jnp.dot(s.reshape(784, 32),
                                            w1_ref[kh, kw],
                                            preferred_element_type=F32)
                acc = jnp.maximum(acc + b1_ref[:], 0.0)
                y_ref[pl.Slice(1 + 28 * t + a, 14, 2),
                      pl.Slice(1 + c, 56, 2), :] = acc.reshape(14, 56, 64)
    for t in range(4):
        p = jnp.zeros((3136, 4), F32)
        for u0 in range(3):
            for v0 in range(3):
                s = y_ref[pl.ds(28 * t + u0, 28), pl.ds(v0, 112), :]
                p = p + jnp.dot(s.reshape(3136, 64), w2_ref[u0, v0],
                                preferred_element_type=F32)
        o_ref[0, pl.ds(3136 * t, 3136), :] = jax.nn.sigmoid(p + b2_ref[:])


def kernel(x, enc1_w, enc1_b, enc2_w, enc2_b, codebook,
           dec1_w, dec1_b, dec2_w, dec2_b):
    B = 16
    xp = jnp.pad(x[:, 0], ((0, 0), (1, 1), (1, 1)))
    xpar = xp.reshape(B, 113, 2, 113, 2).transpose(0, 2, 4, 1, 3)
    taps = [xpar[:, kh % 2, kw % 2, kh // 2:kh // 2 + 112,
                 kw // 2:kw // 2 + 112]
            for kh in range(4) for kw in range(4)]
    xcol = jnp.stack(taps, axis=-1).reshape(B, 12544, 16)
    w1m = enc1_w[:, 0].reshape(64, 16).T
    w2m = enc2_w.transpose(2, 3, 1, 0).reshape(16, 64, 32)
    zf = pl.pallas_call(
        _enc_body,
        grid=(B,),
        in_specs=[pl.BlockSpec((1, 12544, 16), lambda b: (b, 0, 0)),
                  pl.BlockSpec((16, 64), lambda b: (0, 0)),
                  pl.BlockSpec((1, 64), lambda b: (0, 0)),
                  pl.BlockSpec((16, 64, 32), lambda b: (0, 0, 0)),
                  pl.BlockSpec((1, 32), lambda b: (0, 0))],
        out_specs=pl.BlockSpec((1, 3136, 32), lambda b: (b, 0, 0)),
        out_shape=jax.ShapeDtypeStruct((B, 3136, 32), F32),
        scratch_shapes=[pltpu.VMEM((114, 114, 64), F32)],
    )(xcol, w1m, enc1_b.reshape(1, 64), w2m, enc2_b.reshape(1, 32))
    zf = zf.reshape(50176, 32)

    nblk = 196
    q, idx, loss = pl.pallas_call(
        _vq_body,
        grid=(nblk,),
        in_specs=[pl.BlockSpec((256, 32), lambda i: (i, 0)),
                  pl.BlockSpec((1024, 32), lambda i: (0, 0))],
        out_specs=[pl.BlockSpec((256, 32), lambda i: (i, 0)),
                   pl.BlockSpec((1, 1, 256), lambda i: (i, 0, 0)),
                   pl.BlockSpec((1, 1), lambda i: (0, 0),
                                memory_space=pltpu.SMEM)],
        out_shape=[jax.ShapeDtypeStruct((50176, 32), F32),
                   jax.ShapeDtypeStruct((nblk, 1, 256), jnp.int32),
                   jax.ShapeDtypeStruct((1, 1), F32)],
    )(zf, codebook)
    indices = idx.reshape(B, 56, 56)
    vq_loss = (loss[0, 0] * (1.25 / (16 * 32 * 56 * 56))).reshape(())

    qp = jnp.pad(q.reshape(B, 56, 56, 32), ((0, 0), (1, 1), (1, 1), (0, 0)))
    wd1 = dec1_w.transpose(2, 3, 1, 0)
    wd2 = jnp.zeros((3, 3, 64, 4), F32)
    for kh in range(4):
        for kw in range(4):
            wd2 = wd2.at[(kh + 1) // 2, (kw + 1) // 2, :,
                         2 * (kh % 2) + (kw % 2)].set(dec2_w[0, :, kh, kw])
    recon_p = pl.pallas_call(
        _dec_body,
        grid=(B,),
        in_specs=[pl.BlockSpec((1, 58, 58, 32), lambda b: (b, 0, 0, 0)),
                  pl.BlockSpec((4, 4, 32, 64), lambda b: (0, 0, 0, 0)),
                  pl.BlockSpec((1, 64), lambda b: (0, 0)),
                  pl.BlockSpec((3, 3, 64, 4), lambda b: (0, 0, 0, 0)),
                  pl.BlockSpec((1, 1), lambda b: (0, 0))],
        out_specs=pl.BlockSpec((1, 12544, 4), lambda b: (b, 0, 0)),
        out_shape=jax.ShapeDtypeStruct((B, 12544, 4), F32),
        scratch_shapes=[pltpu.VMEM((114, 114, 64), F32)],
    )(qp, wd1, dec1_b.reshape(1, 64), wd2, dec2_b.reshape(1, 1))
    x_recon = recon_p.reshape(B, 112, 112, 2, 2).transpose(0, 1, 3, 2, 4)
    x_recon = x_recon.reshape(B, 1, 224, 224)
    return (x_recon, vq_loss, indices)


# speedup vs baseline: 1.2759x; 1.0551x over previous
import jax
import jax.numpy as jnp
from jax import lax
from jax.experimental import pallas as pl
from jax.experimental.pallas import tpu as pltpu

F32 = jnp.float32


def _zero_halo(ref):
    ref[0:1, :, :] = jnp.zeros((1, 114, 64), F32)
    ref[113:114, :, :] = jnp.zeros((1, 114, 64), F32)
    ref[:, 0:1, :] = jnp.zeros((114, 1, 64), F32)
    ref[:, 113:114, :] = jnp.zeros((114, 1, 64), F32)


def _enc_body(x_ref, w1_ref, b1_ref, w2_ref, b2_ref, o_ref, z1_ref):
    _zero_halo(z1_ref)
    for t in range(8):
        acc = jnp.dot(x_ref[0, pl.ds(1568 * t, 1568), :], w1_ref[:],
                      preferred_element_type=F32)
        acc = jnp.maximum(acc + b1_ref[:], 0.0)
        z1_ref[pl.ds(1 + 14 * t, 14), 1:113, :] = acc.reshape(14, 112, 64)
    for t in range(4):
        acc = jnp.zeros((784, 32), F32)
        for kh in range(4):
            for kw in range(4):
                s = z1_ref[pl.Slice(28 * t + kh, 14, 2),
                           pl.Slice(kw, 56, 2), :]
                acc = acc + jnp.dot(s.reshape(784, 64), w2_ref[kh * 4 + kw],
                                    preferred_element_type=F32)
        o_ref[0, pl.ds(784 * t, 784), :] = acc + b2_ref[:]


def _vq_body(z_ref, c_ref, q_ref, i_ref, l_ref):
    z = z_ref[:]
    c = c_ref[:]
    zn = jnp.sum(z * z, axis=1, keepdims=True)
    cn = jnp.sum(c * c, axis=1)
    m = lax.dot_general(z, c, (((1,), (1,)), ((), ())),
                        preferred_element_type=F32)
    d = (zn + cn[None, :]) - 2.0 * m
    dmin = jnp.min(d, axis=1, keepdims=True)
    iota = lax.broadcasted_iota(jnp.int32, (256, 1024), 1)
    idx = jnp.min(jnp.where(d == dmin, iota, 1024), axis=1)
    onehot = (iota == idx[:, None]).astype(F32)
    q = jnp.dot(onehot, c, preferred_element_type=F32)
    q_ref[:] = q
    i_ref[0, 0, :] = idx
    diff = q - z
    ss = jnp.sum(diff * diff)
    @pl.when(pl.program_id(0) == 0)
    def _():
        l_ref[0, 0] = ss
    @pl.when(pl.program_id(0) != 0)
    def _():
        l_ref[0, 0] = l_ref[0, 0] + ss


def _dec_body(q_ref, w1_ref, b1_ref, w2_ref, b2_ref, o_ref, y_ref):
    _zero_halo(y_ref)
    for a in range(2):
        for c in range(2):
            for t in range(4):
                acc = jnp.zeros((784, 64), F32)
                for dh in range(2):
                    for dw in range(2):
                        kh = a + 2 * dh
                        kw = c + 2 * dw
                        u0 = (kh + 1) // 2
                        v0 = (kw + 1) // 2
                        s = q_ref[0, pl.ds(14 * t + u0, 14),
                                  pl.ds(v0, 56), :]
                        acc = acc + jnp.dot(s.reshape(784, 32),
                                            w1_ref[kh, kw],
                                            preferred_element_type=F32)
                acc = jnp.maximum(acc + b1_ref[:], 0.0)
                y_ref[pl.Slice(1 + 28 * t + a, 14, 2),
                      pl.Slice(1 + c, 56, 2), :] = acc.reshape(14, 56, 64)
    for t in range(4):
        p = jnp.zeros((3136, 4), F32)
        for u0 in range(3):
            for v0 in range(3):
                s = y_ref[pl.ds(28 * t + u0, 28), pl.ds(v0, 112), :]
                p = p + jnp.dot(s.reshape(3136, 64), w2_ref[u0, v0],
                                preferred_element_type=F32)
        o_ref[0, pl.ds(3136 * t, 3136), :] = jax.nn.sigmoid(p + b2_ref[:])


def kernel(x, enc1_w, enc1_b, enc2_w, enc2_b, codebook,
           dec1_w, dec1_b, dec2_w, dec2_b):
    B = 16
    xp = jnp.pad(x[:, 0], ((0, 0), (1, 1), (1, 1)))
    taps = [xp[:, kh:kh + 223:2, kw:kw + 223:2]
            for kh in range(4) for kw in range(4)]
    xcol = jnp.stack(taps, axis=-1).reshape(B, 12544, 16)
    w1m = enc1_w[:, 0].reshape(64, 16).T
    w2m = enc2_w.transpose(2, 3, 1, 0).reshape(16, 64, 32)
    zf = pl.pallas_call(
        _enc_body,
        grid=(B,),
        in_specs=[pl.BlockSpec((1, 12544, 16), lambda b: (b, 0, 0)),
                  pl.BlockSpec((16, 64), lambda b: (0, 0)),
                  pl.BlockSpec((1, 64), lambda b: (0, 0)),
                  pl.BlockSpec((16, 64, 32), lambda b: (0, 0, 0)),
                  pl.BlockSpec((1, 32), lambda b: (0, 0))],
        out_specs=pl.BlockSpec((1, 3136, 32), lambda b: (b, 0, 0)),
        out_shape=jax.ShapeDtypeStruct((B, 3136, 32), F32),
        scratch_shapes=[pltpu.VMEM((114, 114, 64), F32)],
    )(xcol, w1m, enc1_b.reshape(1, 64), w2m, enc2_b.reshape(1, 32))
    zf = zf.reshape(50176, 32)

    nblk = 196
    q, idx, loss = pl.pallas_call(
        _vq_body,
        grid=(nblk,),
        in_specs=[pl.BlockSpec((256, 32), lambda i: (i, 0)),
                  pl.BlockSpec((1024, 32), lambda i: (0, 0))],
        out_specs=[pl.BlockSpec((256, 32), lambda i: (i, 0)),
                   pl.BlockSpec((1, 1, 256), lambda i: (i, 0, 0)),
                   pl.BlockSpec((1, 1), lambda i: (0, 0),
                                memory_space=pltpu.SMEM)],
        out_shape=[jax.ShapeDtypeStruct((50176, 32), F32),
                   jax.ShapeDtypeStruct((nblk, 1, 256), jnp.int32),
                   jax.ShapeDtypeStruct((1, 1), F32)],
    )(zf, codebook)
    indices = idx.reshape(B, 56, 56)
    vq_loss = (loss[0, 0] * (1.25 / (16 * 32 * 56 * 56))).reshape(())

    qp = jnp.pad(q.reshape(B, 56, 56, 32), ((0, 0), (1, 1), (1, 1), (0, 0)))
    wd1 = dec1_w.transpose(2, 3, 1, 0)
    zcol = jnp.zeros((64,), F32)
    rows = []
    for u0 in range(3):
        cols = []
        for v0 in range(3):
            mats = []
            for a in range(2):
                kh = 2 * u0 - a
                for c in range(2):
                    kw = 2 * v0 - c
                    ok = 0 <= kh <= 3 and 0 <= kw <= 3
                    mats.append(dec2_w[0, :, kh, kw] if ok else zcol)
            cols.append(jnp.stack(mats, axis=-1))
        rows.append(jnp.stack(cols))
    wd2 = jnp.stack(rows)
    recon_p = pl.pallas_call(
        _dec_body,
        grid=(B,),
        in_specs=[pl.BlockSpec((1, 58, 58, 32), lambda b: (b, 0, 0, 0)),
                  pl.BlockSpec((4, 4, 32, 64), lambda b: (0, 0, 0, 0)),
                  pl.BlockSpec((1, 64), lambda b: (0, 0)),
                  pl.BlockSpec((3, 3, 64, 4), lambda b: (0, 0, 0, 0)),
                  pl.BlockSpec((1, 1), lambda b: (0, 0))],
        out_specs=pl.BlockSpec((1, 12544, 4), lambda b: (b, 0, 0)),
        out_shape=jax.ShapeDtypeStruct((B, 12544, 4), F32),
        scratch_shapes=[pltpu.VMEM((114, 114, 64), F32)],
    )(qp, wd1, dec1_b.reshape(1, 64), wd2, dec2_b.reshape(1, 1))
    x_recon = recon_p.reshape(B, 112, 112, 2, 2).transpose(0, 1, 3, 2, 4)
    x_recon = x_recon.reshape(B, 1, 224, 224)
    return (x_recon, vq_loss, indices)


# speedup vs baseline: 1.2950x; 1.0149x over previous
import jax
import jax.numpy as jnp
from jax import lax
from jax.experimental import pallas as pl
from jax.experimental.pallas import tpu as pltpu

F32 = jnp.float32


def _zero_halo(ref):
    ref[0:1, :, :] = jnp.zeros((1, 114, 64), F32)
    ref[113:114, :, :] = jnp.zeros((1, 114, 64), F32)
    ref[:, 0:1, :] = jnp.zeros((114, 1, 64), F32)
    ref[:, 113:114, :] = jnp.zeros((114, 1, 64), F32)


def _enc_body(x_ref, w1_ref, b1_ref, w2_ref, b2_ref, o_ref, z1_ref):
    _zero_halo(z1_ref)
    for t in range(8):
        acc = jnp.dot(x_ref[0, pl.ds(1568 * t, 1568), :], w1_ref[:],
                      preferred_element_type=F32)
        acc = jnp.maximum(acc + b1_ref[:], 0.0)
        z1_ref[pl.ds(1 + 14 * t, 14), 1:113, :] = acc.reshape(14, 112, 64)
    for t in range(4):
        acc = jnp.zeros((784, 32), F32)
        for kh in range(4):
            for kw in range(4):
                s = z1_ref[pl.Slice(28 * t + kh, 14, 2),
                           pl.Slice(kw, 56, 2), :]
                acc = acc + jnp.dot(s.reshape(784, 64), w2_ref[kh * 4 + kw],
                                    preferred_element_type=F32)
        o_ref[0, pl.ds(784 * t, 784), :] = acc + b2_ref[:]


def _vq_body(z_ref, c_ref, q_ref, i_ref, l_ref):
    z = z_ref[:]
    c = c_ref[:]
    zn = jnp.sum(z * z, axis=1, keepdims=True)
    cn = jnp.sum(c * c, axis=1)
    m = lax.dot_general(z, c, (((1,), (1,)), ((), ())),
                        preferred_element_type=F32)
    d = (zn + cn[None, :]) - 2.0 * m
    dmin = jnp.min(d, axis=1, keepdims=True)
    iota = lax.broadcasted_iota(jnp.int32, (512, 1024), 1)
    idx = jnp.min(jnp.where(d == dmin, iota, 1024), axis=1)
    onehot = (iota == idx[:, None]).astype(F32)
    q = jnp.dot(onehot, c, preferred_element_type=F32)
    q_ref[:] = q
    i_ref[0, 0, :] = idx
    diff = q - z
    ss = jnp.sum(diff * diff)
    @pl.when(pl.program_id(0) == 0)
    def _():
        l_ref[0, 0] = ss
    @pl.when(pl.program_id(0) != 0)
    def _():
        l_ref[0, 0] = l_ref[0, 0] + ss


def _dec_body(q_ref, w1_ref, b1_ref, w2_ref, b2_ref, o_ref, y_ref):
    _zero_halo(y_ref)
    for a in range(2):
        for c in range(2):
            for t in range(4):
                acc = jnp.zeros((784, 64), F32)
                for dh in range(2):
                    for dw in range(2):
                        kh = a + 2 * dh
                        kw = c + 2 * dw
                        u0 = (kh + 1) // 2
                        v0 = (kw + 1) // 2
                        s = q_ref[0, pl.ds(14 * t + u0, 14),
                                  pl.ds(v0, 56), :]
                        acc = acc + jnp.dot(s.reshape(784, 32),
                                            w1_ref[kh, kw],
                                            preferred_element_type=F32)
                acc = jnp.maximum(acc + b1_ref[:], 0.0)
                y_ref[pl.Slice(1 + 28 * t + a, 14, 2),
                      pl.Slice(1 + c, 56, 2), :] = acc.reshape(14, 56, 64)
    for t in range(4):
        p = jnp.zeros((3136, 4), F32)
        for u0 in range(3):
            for v0 in range(3):
                s = y_ref[pl.ds(28 * t + u0, 28), pl.ds(v0, 112), :]
                p = p + jnp.dot(s.reshape(3136, 64), w2_ref[u0, v0],
                                preferred_element_type=F32)
        o_ref[0, pl.ds(3136 * t, 3136), :] = jax.nn.sigmoid(p + b2_ref[:])


def kernel(x, enc1_w, enc1_b, enc2_w, enc2_b, codebook,
           dec1_w, dec1_b, dec2_w, dec2_b):
    B = 16
    xp = jnp.pad(x[:, 0], ((0, 0), (1, 1), (1, 1)))
    taps = [xp[:, kh:kh + 223:2, kw:kw + 223:2]
            for kh in range(4) for kw in range(4)]
    xcol = jnp.stack(taps, axis=-1).reshape(B, 12544, 16)
    w1m = enc1_w[:, 0].reshape(64, 16).T
    w2m = enc2_w.transpose(2, 3, 1, 0).reshape(16, 64, 32)
    zf = pl.pallas_call(
        _enc_body,
        grid=(B,),
        in_specs=[pl.BlockSpec((1, 12544, 16), lambda b: (b, 0, 0)),
                  pl.BlockSpec((16, 64), lambda b: (0, 0)),
                  pl.BlockSpec((1, 64), lambda b: (0, 0)),
                  pl.BlockSpec((16, 64, 32), lambda b: (0, 0, 0)),
                  pl.BlockSpec((1, 32), lambda b: (0, 0))],
        out_specs=pl.BlockSpec((1, 3136, 32), lambda b: (b, 0, 0)),
        out_shape=jax.ShapeDtypeStruct((B, 3136, 32), F32),
        scratch_shapes=[pltpu.VMEM((114, 114, 64), F32)],
    )(xcol, w1m, enc1_b.reshape(1, 64), w2m, enc2_b.reshape(1, 32))
    zf = zf.reshape(50176, 32)

    nblk = 98
    q, idx, loss = pl.pallas_call(
        _vq_body,
        grid=(nblk,),
        in_specs=[pl.BlockSpec((512, 32), lambda i: (i, 0)),
                  pl.BlockSpec((1024, 32), lambda i: (0, 0))],
        out_specs=[pl.BlockSpec((512, 32), lambda i: (i, 0)),
                   pl.BlockSpec((1, 1, 512), lambda i: (i, 0, 0)),
                   pl.BlockSpec((1, 1), lambda i: (0, 0),
                                memory_space=pltpu.SMEM)],
        out_shape=[jax.ShapeDtypeStruct((50176, 32), F32),
                   jax.ShapeDtypeStruct((nblk, 1, 512), jnp.int32),
                   jax.ShapeDtypeStruct((1, 1), F32)],
    )(zf, codebook)
    indices = idx.reshape(B, 56, 56)
    vq_loss = (loss[0, 0] * (1.25 / (16 * 32 * 56 * 56))).reshape(())

    qp = jnp.pad(q.reshape(B, 56, 56, 32), ((0, 0), (1, 1), (1, 1), (0, 0)))
    wd1 = dec1_w.transpose(2, 3, 1, 0)
    zcol = jnp.zeros((64,), F32)
    rows = []
    for u0 in range(3):
        cols = []
        for v0 in range(3):
            mats = []
            for a in range(2):
                kh = 2 * u0 - a
                for c in range(2):
                    kw = 2 * v0 - c
                    ok = 0 <= kh <= 3 and 0 <= kw <= 3
                    mats.append(dec2_w[0, :, kh, kw] if ok else zcol)
            cols.append(jnp.stack(mats, axis=-1))
        rows.append(jnp.stack(cols))
    wd2 = jnp.stack(rows)
    recon_p = pl.pallas_call(
        _dec_body,
        grid=(B,),
        in_specs=[pl.BlockSpec((1, 58, 58, 32), lambda b: (b, 0, 0, 0)),
                  pl.BlockSpec((4, 4, 32, 64), lambda b: (0, 0, 0, 0)),
                  pl.BlockSpec((1, 64), lambda b: (0, 0)),
                  pl.BlockSpec((3, 3, 64, 4), lambda b: (0, 0, 0, 0)),
                  pl.BlockSpec((1, 1), lambda b: (0, 0))],
        out_specs=pl.BlockSpec((1, 12544, 4), lambda b: (b, 0, 0)),
        out_shape=jax.ShapeDtypeStruct((B, 12544, 4), F32),
        scratch_shapes=[pltpu.VMEM((114, 114, 64), F32)],
    )(qp, wd1, dec1_b.reshape(1, 64), wd2, dec2_b.reshape(1, 1))
    x_recon = recon_p.reshape(B, 112, 112, 2, 2).transpose(0, 1, 3, 2, 4)
    x_recon = x_recon.reshape(B, 1, 224, 224)
    return (x_recon, vq_loss, indices)


# speedup vs baseline: 1.3006x; 1.0043x over previous
import jax
import jax.numpy as jnp
from jax import lax
from jax.experimental import pallas as pl
from jax.experimental.pallas import tpu as pltpu

F32 = jnp.float32


def _zero_halo(ref):
    ref[0:1, :, :] = jnp.zeros((1, 114, 64), F32)
    ref[113:114, :, :] = jnp.zeros((1, 114, 64), F32)
    ref[:, 0:1, :] = jnp.zeros((114, 1, 64), F32)
    ref[:, 113:114, :] = jnp.zeros((114, 1, 64), F32)


def _enc_body(x_ref, w1_ref, b1_ref, w2_ref, b2_ref, o_ref, z1_ref):
    _zero_halo(z1_ref)
    for t in range(8):
        acc = jnp.dot(x_ref[0, pl.ds(1568 * t, 1568), :], w1_ref[:],
                      preferred_element_type=F32)
        acc = jnp.maximum(acc + b1_ref[:], 0.0)
        z1_ref[pl.ds(1 + 14 * t, 14), 1:113, :] = acc.reshape(14, 112, 64)
    for t in range(4):
        acc = jnp.zeros((784, 32), F32)
        for kh in range(4):
            for kw in range(4):
                s = z1_ref[pl.Slice(28 * t + kh, 14, 2),
                           pl.Slice(kw, 56, 2), :]
                acc = acc + jnp.dot(s.reshape(784, 64), w2_ref[kh * 4 + kw],
                                    preferred_element_type=F32)
        o_ref[0, pl.ds(784 * t, 784), :] = acc + b2_ref[:]


def _vq_body(z_ref, c_ref, q_ref, i_ref, l_ref):
    z = z_ref[:]
    c = c_ref[:]
    zn = jnp.sum(z * z, axis=1, keepdims=True)
    cn = jnp.sum(c * c, axis=1)
    m = lax.dot_general(z, c, (((1,), (1,)), ((), ())),
                        preferred_element_type=F32)
    d = (zn + cn[None, :]) - 2.0 * m
    dmin = jnp.min(d, axis=1, keepdims=True)
    iota = lax.broadcasted_iota(jnp.int32, (512, 1024), 1)
    idx = jnp.min(jnp.where(d == dmin, iota, 1024), axis=1)
    onehot = (iota == idx[:, None]).astype(F32)
    q = jnp.dot(onehot, c, preferred_element_type=F32)
    q_ref[:] = q
    i_ref[0, 0, :] = idx
    diff = q - z
    ss = jnp.sum(diff * diff)
    @pl.when(pl.program_id(0) == 0)
    def _():
        l_ref[0, 0] = ss
    @pl.when(pl.program_id(0) != 0)
    def _():
        l_ref[0, 0] = l_ref[0, 0] + ss


def _dec_body(q_ref, w1_ref, b1_ref, w2_ref, b2_ref, o_ref, y_ref):
    _zero_halo(y_ref)
    for a in range(2):
        for c in range(2):
            for t in range(4):
                acc = jnp.zeros((784, 64), F32)
                for dh in range(2):
                    for dw in range(2):
                        kh = a + 2 * dh
                        kw = c + 2 * dw
                        u0 = (kh + 1) // 2
                        v0 = (kw + 1) // 2
                        s = q_ref[0, pl.ds(14 * t + u0, 14),
                                  pl.ds(v0, 56), :]
                        acc = acc + jnp.dot(s.reshape(784, 32),
                                            w1_ref[kh, kw],
                                            preferred_element_type=F32)
                acc = jnp.maximum(acc + b1_ref[:], 0.0)
                y_ref[pl.Slice(1 + 28 * t + a, 14, 2),
                      pl.Slice(1 + c, 56, 2), :] = acc.reshape(14, 56, 64)
    for t in range(4):
        p = jnp.zeros((3136, 4), F32)
        for u0 in range(3):
            for v0 in range(3):
                s = y_ref[pl.ds(28 * t + u0, 28), pl.ds(v0, 112), :]
                p = p + jnp.dot(s.reshape(3136, 64), w2_ref[u0, v0],
                                preferred_element_type=F32)
        o_ref[0, pl.ds(3136 * t, 3136), :] = jax.nn.sigmoid(p + b2_ref[:])


def kernel(x, enc1_w, enc1_b, enc2_w, enc2_b, codebook,
           dec1_w, dec1_b, dec2_w, dec2_b):
    B = 16
    xp = jnp.pad(x[:, 0], ((0, 0), (1, 1), (1, 1)))
    taps = [xp[:, kh:kh + 223:2, kw:kw + 223:2]
            for kh in range(4) for kw in range(4)]
    xcol = jnp.stack(taps, axis=-1).reshape(B, 12544, 16)
    w1m = enc1_w[:, 0].reshape(64, 16).T
    w2m = enc2_w.transpose(2, 3, 1, 0).reshape(16, 64, 32)
    zf = pl.pallas_call(
        _enc_body,
        grid=(B,),
        in_specs=[pl.BlockSpec((1, 12544, 16), lambda b: (b, 0, 0)),
                  pl.BlockSpec((16, 64), lambda b: (0, 0)),
                  pl.BlockSpec((1, 64), lambda b: (0, 0)),
                  pl.BlockSpec((16, 64, 32), lambda b: (0, 0, 0)),
                  pl.BlockSpec((1, 32), lambda b: (0, 0))],
        out_specs=pl.BlockSpec((1, 3136, 32), lambda b: (b, 0, 0)),
        out_shape=jax.ShapeDtypeStruct((B, 3136, 32), F32),
        scratch_shapes=[pltpu.VMEM((114, 114, 64), F32)],
        compiler_params=pltpu.CompilerParams(
            dimension_semantics=("parallel",)),
    )(xcol, w1m, enc1_b.reshape(1, 64), w2m, enc2_b.reshape(1, 32))
    zf = zf.reshape(50176, 32)

    nblk = 98
    q, idx, loss = pl.pallas_call(
        _vq_body,
        grid=(nblk,),
        in_specs=[pl.BlockSpec((512, 32), lambda i: (i, 0)),
                  pl.BlockSpec((1024, 32), lambda i: (0, 0))],
        out_specs=[pl.BlockSpec((512, 32), lambda i: (i, 0)),
                   pl.BlockSpec((1, 1, 512), lambda i: (i, 0, 0)),
                   pl.BlockSpec((1, 1), lambda i: (0, 0),
                                memory_space=pltpu.SMEM)],
        out_shape=[jax.ShapeDtypeStruct((50176, 32), F32),
                   jax.ShapeDtypeStruct((nblk, 1, 512), jnp.int32),
                   jax.ShapeDtypeStruct((1, 1), F32)],
    )(zf, codebook)
    indices = idx.reshape(B, 56, 56)
    vq_loss = (loss[0, 0] * (1.25 / (16 * 32 * 56 * 56))).reshape(())

    qp = jnp.pad(q.reshape(B, 56, 56, 32), ((0, 0), (1, 1), (1, 1), (0, 0)))
    wd1 = dec1_w.transpose(2, 3, 1, 0)
    zcol = jnp.zeros((64,), F32)
    rows = []
    for u0 in range(3):
        cols = []
        for v0 in range(3):
            mats = []
            for a in range(2):
                kh = 2 * u0 - a
                for c in range(2):
                    kw = 2 * v0 - c
                    ok = 0 <= kh <= 3 and 0 <= kw <= 3
                    mats.append(dec2_w[0, :, kh, kw] if ok else zcol)
            cols.append(jnp.stack(mats, axis=-1))
        rows.append(jnp.stack(cols))
    wd2 = jnp.stack(rows)
    recon_p = pl.pallas_call(
        _dec_body,
        grid=(B,),
        in_specs=[pl.BlockSpec((1, 58, 58, 32), lambda b: (b, 0, 0, 0)),
                  pl.BlockSpec((4, 4, 32, 64), lambda b: (0, 0, 0, 0)),
                  pl.BlockSpec((1, 64), lambda b: (0, 0)),
                  pl.BlockSpec((3, 3, 64, 4), lambda b: (0, 0, 0, 0)),
                  pl.BlockSpec((1, 1), lambda b: (0, 0))],
        out_specs=pl.BlockSpec((1, 12544, 4), lambda b: (b, 0, 0)),
        out_shape=jax.ShapeDtypeStruct((B, 12544, 4), F32),
        scratch_shapes=[pltpu.VMEM((114, 114, 64), F32)],
        compiler_params=pltpu.CompilerParams(
            dimension_semantics=("parallel",)),
    )(qp, wd1, dec1_b.reshape(1, 64), wd2, dec2_b.reshape(1, 1))
    x_recon = recon_p.reshape(B, 112, 112, 2, 2).transpose(0, 1, 3, 2, 4)
    x_recon = x_recon.reshape(B, 1, 224, 224)
    return (x_recon, vq_loss, indices)
